# Initial kernel scaffold; baseline (speedup 1.0000x reference)
import jax
import jax.numpy as jnp
from jax.experimental import pallas as pl


def kernel(x, Wg, bg, We, be):
    raise NotImplementedError("write your pallas kernel here")


# speedup vs baseline: 1.8723x; 1.8723x over previous
import jax
import jax.numpy as jnp
from jax.experimental import pallas as pl
from jax.experimental.pallas import tpu as pltpu

_BN = 1024
_BF = 1024


def _moe_body(x_ref, wg_ref, bg_ref, we_ref, be_ref, out_ref, g_scr, xb_scr):
    f = pl.program_id(1)
    e = pl.program_id(2)
    n_exp = g_scr.shape[1]

    @pl.when((f == 0) & (e == 0))
    def _prep():
        xf = x_ref[...]
        logits = jnp.dot(xf, wg_ref[...], preferred_element_type=jnp.float32)
        logits = logits + bg_ref[...]
        m = jnp.max(logits, axis=-1, keepdims=True)
        p = jnp.exp(logits - m)
        g_scr[...] = p / jnp.sum(p, axis=-1, keepdims=True)
        xb_scr[...] = xf.astype(jnp.bfloat16)

    lane = jax.lax.broadcasted_iota(jnp.int32, (1, n_exp), 1)
    ge = jnp.sum(jnp.where(lane == e, g_scr[...], 0.0), axis=-1, keepdims=True)

    mm = jnp.dot(xb_scr[...], we_ref[0].astype(jnp.bfloat16),
                 preferred_element_type=jnp.float32)
    contrib = ge * (mm + be_ref[0])

    @pl.when(e == 0)
    def _init():
        out_ref[...] = contrib

    @pl.when(e != 0)
    def _acc():
        out_ref[...] += contrib


def kernel(x, Wg, bg, We, be):
    n, k = x.shape
    n_exp = Wg.shape[1]
    f_out = We.shape[2]
    bn = min(_BN, n)
    bf = min(_BF, f_out)
    grid = (n // bn, f_out // bf, n_exp)
    return pl.pallas_call(
        _moe_body,
        grid=grid,
        in_specs=[
            pl.BlockSpec((bn, k), lambda i, f, e: (i, 0)),
            pl.BlockSpec((k, n_exp), lambda i, f, e: (0, 0)),
            pl.BlockSpec((1, n_exp), lambda i, f, e: (0, 0)),
            pl.BlockSpec((1, k, bf), lambda i, f, e: (e, 0, f)),
            pl.BlockSpec((1, 1, bf), lambda i, f, e: (e, 0, f)),
        ],
        out_specs=pl.BlockSpec((bn, bf), lambda i, f, e: (i, f)),
        out_shape=jax.ShapeDtypeStruct((n, f_out), jnp.float32),
        scratch_shapes=[
            pltpu.VMEM((bn, n_exp), jnp.float32),
            pltpu.VMEM((bn, k), jnp.bfloat16),
        ],
        compiler_params=pltpu.CompilerParams(
            dimension_semantics=("parallel", "parallel", "arbitrary"),
        ),
    )(x, Wg, bg.reshape(1, n_exp), We, be.reshape(n_exp, 1, f_out))
